# baseline (speedup 1.0000x reference)
import functools

import jax
import jax.numpy as jnp
from jax import lax
from jax.experimental import pallas as pl
from jax.experimental.pallas import tpu as pltpu
from jax.experimental.pallas import tpu_sc as plsc

B = 16384
D = 64
UNITS = 48
STATE = 1000
H1 = 24
NW = 32
BPW = B // NW
NCH = BPW // 128
SPAD = 1024
BLK = 2048
HEXT = B + BLK
LW = 128

_mesh = plsc.VectorSubcoreMesh(core_axis_name="c", subcore_axis_name="s")
_sc_params = pltpu.CompilerParams(needs_layout_passes=False)


def _wid():
    return lax.axis_index("s") * 2 + lax.axis_index("c")


def _gather_body(ids_hbm, mem_hbm, h_hbm, win_hbm, idx_v, rows_v, wtab_v,
                 sem, wsem):
    wid = _wid()
    base = wid * BPW
    idx_loads = [
        pltpu.async_copy(ids_hbm.at[pl.ds(base + k * 128, 128)], idx_v.at[k],
                         wsem)
        for k in range(NCH)
    ]
    gathers = []
    for k in range(NCH):
        idx_loads[k].wait()
        gathers.append(
            pltpu.async_copy(mem_hbm.at[idx_v.at[k]],
                             rows_v.at[pl.ds(k * 128, 128)], sem))
    for g in gathers:
        g.wait()
    hw = pltpu.async_copy(rows_v, h_hbm.at[pl.ds(base, BPW)], wsem)
    neg1 = jnp.full((16,), -1, jnp.int32)
    for g in range(SPAD // 16):
        wtab_v[pl.ds(g * 16, 16)] = neg1
    iota = lax.iota(jnp.int32, 16)
    for k in range(NCH):
        for j in range(128 // 16):
            idv = idx_v[k, pl.ds(j * 16, 16)]
            vals = jnp.full((16,), base + k * 128 + j * 16, jnp.int32) + iota
            _, last = plsc.scan_count(idv)
            plsc.store_scatter(wtab_v, (idv,), vals, mask=last)
    pltpu.sync_copy(wtab_v, win_hbm.at[wid])
    hw.wait()


@functools.partial(
    pl.kernel,
    out_type=[
        jax.ShapeDtypeStruct((B, LW), jnp.float32),
        jax.ShapeDtypeStruct((NW, SPAD), jnp.int32),
    ],
    mesh=_mesh,
    scratch_types=[
        pltpu.VMEM((NCH, 128), jnp.int32),
        pltpu.VMEM((BPW, LW), jnp.float32),
        pltpu.VMEM((SPAD,), jnp.int32),
        pltpu.SemaphoreType.DMA,
        pltpu.SemaphoreType.DMA,
    ],
    compiler_params=_sc_params,
)
def _sc_gather(ids_hbm, mem_hbm, h_hbm, win_hbm, idx_v, rows_v, wtab_v,
               sem, wsem):
    _gather_body(ids_hbm, mem_hbm, h_hbm, win_hbm, idx_v, rows_v, wtab_v,
                 sem, wsem)


def _tc_body(x_ref, h_ref, mem_ref, wcat_ref, uh_ref, w1h_ref, bcat_ref,
             b1_ref, w2r_ref, b2_ref, out_ref, hnew_ref):
    i = pl.program_id(0)
    f32 = jnp.float32

    @pl.when(i < B // BLK)
    def _dense():
        x = x_ref[...]
        h = h_ref[:, :UNITS]
        xh = jnp.concatenate([x, h], axis=1)
        p1 = jnp.dot(xh, wcat_ref[...], preferred_element_type=f32) + bcat_ref[...]
        zr = jax.nn.sigmoid(p1[:, :2 * UNITS])
        z = zr[:, :UNITS]
        r = zr[:, UNITS:]
        hh = jnp.tanh(p1[:, 2 * UNITS:3 * UNITS]
                      + jnp.dot(r * h, uh_ref[...], preferred_element_type=f32))
        h_new = z * h + (1.0 - z) * hh
        hdn = jax.nn.relu(p1[:, 3 * UNITS:]
                          + jnp.dot(h_new, w1h_ref[...], preferred_element_type=f32)
                          + b1_ref[...])
        out_ref[...] = jax.nn.sigmoid(
            jnp.dot(hdn, w2r_ref[...], preferred_element_type=f32) + b2_ref[...])
        hnew_ref[...] = jnp.concatenate(
            [h_new, jnp.zeros((BLK, LW - UNITS), f32)], axis=1)

    @pl.when(i == B // BLK)
    def _append_mem():
        hnew_ref[pl.ds(0, STATE), :] = mem_ref[...]
        hnew_ref[pl.ds(STATE, SPAD - STATE), :] = mem_ref[pl.ds(0, SPAD - STATE), :]
        hnew_ref[pl.ds(SPAD, BLK - SPAD), :] = mem_ref[pl.ds(0, BLK - SPAD), :]


def _dense_stage(inputs, h, mem128, kernel, rec_kernel, bias, W1, b1, W2, b2):
    top = jnp.concatenate([kernel, W1[:D]], axis=1)
    bot = jnp.concatenate(
        [rec_kernel[:, :2 * UNITS],
         jnp.zeros((UNITS, UNITS + H1), jnp.float32)], axis=1)
    wcat = jnp.concatenate([top, bot], axis=0)
    uh = rec_kernel[:, 2 * UNITS:]
    w1h = W1[D:]
    bcat = jnp.concatenate([bias, jnp.zeros((H1,), jnp.float32)]).reshape(1, -1)
    b1r = b1.reshape(1, H1)
    b2r = b2.reshape(1, 1)

    nblk = B // BLK
    full = lambda shape: pl.BlockSpec(shape, lambda i: (0, 0))
    out, hnew_ext = pl.pallas_call(
        _tc_body,
        grid=(nblk + 1,),
        in_specs=[
            pl.BlockSpec((BLK, D), lambda i: (jnp.minimum(i, nblk - 1), 0)),
            pl.BlockSpec((BLK, LW), lambda i: (jnp.minimum(i, nblk - 1), 0)),
            full((STATE, LW)),
            full((D + UNITS, 3 * UNITS + H1)),
            full((UNITS, UNITS)),
            full((UNITS, H1)),
            full((1, 3 * UNITS + H1)),
            full((1, H1)),
            full((H1, 1)),
            full((1, 1)),
        ],
        out_specs=[
            pl.BlockSpec((BLK, 1), lambda i: (jnp.minimum(i, nblk - 1), 0)),
            pl.BlockSpec((BLK, LW), lambda i: (i, 0)),
        ],
        out_shape=[
            jax.ShapeDtypeStruct((B, 1), jnp.float32),
            jax.ShapeDtypeStruct((HEXT, LW), jnp.float32),
        ],
    )(inputs, h, mem128, wcat, uh, w1h, bcat, b1r, W2, b2r)
    return out, hnew_ext


NSW = 128
NWS = SPAD // NSW
TAIL = STATE - (NWS - 1) * NSW


def _scatter_body(win_hbm, hnew_hbm, newmem_hbm, wbuf_v, idx_v, rows_v, sem):
    wid = _wid()

    @pl.when(wid < NWS)
    def _go():
        s0 = wid * NSW
        pltpu.sync_copy(win_hbm.at[:, pl.ds(s0, NSW)], wbuf_v)
        iota = lax.iota(jnp.int32, 16)
        for g in range(NSW // 16):
            m = wbuf_v[0, pl.ds(g * 16, 16)]
            for t in range(1, NW):
                m = jnp.maximum(m, wbuf_v[t, pl.ds(g * 16, 16)])
            fallback = jnp.full((16,), B + s0 + g * 16, jnp.int32) + iota
            idx_v[pl.ds(g * 16, 16)] = jnp.where(m < 0, fallback, m)
        pltpu.async_copy(hnew_hbm.at[idx_v], rows_v, sem).wait()

        @pl.when(wid < NWS - 1)
        def _full():
            pltpu.sync_copy(rows_v, newmem_hbm.at[pl.ds(s0, NSW)])

        @pl.when(wid == NWS - 1)
        def _tail():
            pltpu.sync_copy(rows_v.at[pl.ds(0, TAIL)],
                            newmem_hbm.at[pl.ds((NWS - 1) * NSW, TAIL)])


@functools.partial(
    pl.kernel,
    out_type=jax.ShapeDtypeStruct((STATE, LW), jnp.float32),
    mesh=_mesh,
    scratch_types=[
        pltpu.VMEM((NW, NSW), jnp.int32),
        pltpu.VMEM((NSW,), jnp.int32),
        pltpu.VMEM((NSW, LW), jnp.float32),
        pltpu.SemaphoreType.DMA,
    ],
    compiler_params=_sc_params,
)
def _sc_scatter(win_hbm, hnew_hbm, newmem_hbm, wbuf_v, idx_v, rows_v, sem):
    _scatter_body(win_hbm, hnew_hbm, newmem_hbm, wbuf_v, idx_v, rows_v, sem)


def kernel(inputs, mem, kernel, rec_kernel, bias, W1, b1, W2, b2):
    ids = jnp.clip(inputs[:, 0].astype(jnp.int32), 0, STATE - 1)
    mem128 = jnp.pad(mem, ((0, 0), (0, LW - UNITS)))
    h, winners = _sc_gather(ids, mem128)
    out, hnew_ext = _dense_stage(inputs, h, mem128, kernel, rec_kernel, bias,
                                 W1, b1, W2, b2)
    new_mem = _sc_scatter(winners, hnew_ext)[:, :UNITS]
    return out, new_mem


# speedup vs baseline: 1.0555x; 1.0555x over previous
import functools

import jax
import jax.numpy as jnp
from jax import lax
from jax.experimental import pallas as pl
from jax.experimental.pallas import tpu as pltpu
from jax.experimental.pallas import tpu_sc as plsc

B = 16384
D = 64
UNITS = 48
STATE = 1000
H1 = 24
NW = 32
BPW = B // NW
NCH = BPW // 128
SPAD = 1024
BLK = 2048
HEXT = B + BLK
LW = 128

_mesh = plsc.VectorSubcoreMesh(core_axis_name="c", subcore_axis_name="s")
_sc_params = pltpu.CompilerParams(needs_layout_passes=False)


def _wid():
    return lax.axis_index("s") * 2 + lax.axis_index("c")


def _gather_body(ids_hbm, mem_hbm, h_hbm, win_hbm, idx_v, rows_v, wtab_v,
                 mem_sp, sem, wsem):
    wid = _wid()
    base = wid * BPW
    idx_loads = [
        pltpu.async_copy(ids_hbm.at[pl.ds(base + k * 128, 128)], idx_v.at[k],
                         wsem)
        for k in range(NCH)
    ]
    @pl.when(lax.axis_index("s") == 0)
    def _stage():
        pltpu.sync_copy(mem_hbm, mem_sp)

    plsc.subcore_barrier()
    gathers = []
    for k in range(NCH):
        idx_loads[k].wait()
        gathers.append(
            pltpu.async_copy(mem_sp.at[idx_v.at[k]],
                             rows_v.at[pl.ds(k * 128, 128)], sem))
    for g in gathers:
        g.wait()
    hw = pltpu.async_copy(rows_v, h_hbm.at[pl.ds(base, BPW)], wsem)
    neg1 = jnp.full((16,), -1, jnp.int32)
    for g in range(SPAD // 16):
        wtab_v[pl.ds(g * 16, 16)] = neg1
    iota = lax.iota(jnp.int32, 16)
    for k in range(NCH):
        for j in range(128 // 16):
            idv = idx_v[k, pl.ds(j * 16, 16)]
            vals = jnp.full((16,), base + k * 128 + j * 16, jnp.int32) + iota
            _, last = plsc.scan_count(idv)
            plsc.store_scatter(wtab_v, (idv,), vals, mask=last)
    pltpu.sync_copy(wtab_v, win_hbm.at[wid])
    hw.wait()


@functools.partial(
    pl.kernel,
    out_type=[
        jax.ShapeDtypeStruct((B, LW), jnp.float32),
        jax.ShapeDtypeStruct((NW, SPAD), jnp.int32),
    ],
    mesh=_mesh,
    scratch_types=[
        pltpu.VMEM((NCH, 128), jnp.int32),
        pltpu.VMEM((BPW, LW), jnp.float32),
        pltpu.VMEM((SPAD,), jnp.int32),
        pltpu.VMEM_SHARED((STATE, LW), jnp.float32),
        pltpu.SemaphoreType.DMA,
        pltpu.SemaphoreType.DMA,
    ],
    compiler_params=_sc_params,
)
def _sc_gather(ids_hbm, mem_hbm, h_hbm, win_hbm, idx_v, rows_v, wtab_v,
               mem_sp, sem, wsem):
    _gather_body(ids_hbm, mem_hbm, h_hbm, win_hbm, idx_v, rows_v, wtab_v,
                 mem_sp, sem, wsem)


def _tc_body(x_ref, h_ref, mem_ref, wcat_ref, uh_ref, w1h_ref, bcat_ref,
             b1_ref, w2r_ref, b2_ref, out_ref, hnew_ref):
    i = pl.program_id(0)
    f32 = jnp.float32

    @pl.when(i < B // BLK)
    def _dense():
        x = x_ref[...]
        h = h_ref[:, :UNITS]
        xh = jnp.concatenate([x, h], axis=1)
        p1 = jnp.dot(xh, wcat_ref[...], preferred_element_type=f32) + bcat_ref[...]
        zr = jax.nn.sigmoid(p1[:, :2 * UNITS])
        z = zr[:, :UNITS]
        r = zr[:, UNITS:]
        hh = jnp.tanh(p1[:, 2 * UNITS:3 * UNITS]
                      + jnp.dot(r * h, uh_ref[...], preferred_element_type=f32))
        h_new = z * h + (1.0 - z) * hh
        hdn = jax.nn.relu(p1[:, 3 * UNITS:]
                          + jnp.dot(h_new, w1h_ref[...], preferred_element_type=f32)
                          + b1_ref[...])
        out_ref[...] = jax.nn.sigmoid(
            jnp.dot(hdn, w2r_ref[...], preferred_element_type=f32) + b2_ref[...])
        hnew_ref[...] = jnp.concatenate(
            [h_new, jnp.zeros((BLK, LW - UNITS), f32)], axis=1)

    @pl.when(i == B // BLK)
    def _append_mem():
        hnew_ref[pl.ds(0, STATE), :] = mem_ref[...]
        hnew_ref[pl.ds(STATE, SPAD - STATE), :] = mem_ref[pl.ds(0, SPAD - STATE), :]
        hnew_ref[pl.ds(SPAD, BLK - SPAD), :] = mem_ref[pl.ds(0, BLK - SPAD), :]


def _dense_stage(inputs, h, mem128, kernel, rec_kernel, bias, W1, b1, W2, b2):
    top = jnp.concatenate([kernel, W1[:D]], axis=1)
    bot = jnp.concatenate(
        [rec_kernel[:, :2 * UNITS],
         jnp.zeros((UNITS, UNITS + H1), jnp.float32)], axis=1)
    wcat = jnp.concatenate([top, bot], axis=0)
    uh = rec_kernel[:, 2 * UNITS:]
    w1h = W1[D:]
    bcat = jnp.concatenate([bias, jnp.zeros((H1,), jnp.float32)]).reshape(1, -1)
    b1r = b1.reshape(1, H1)
    b2r = b2.reshape(1, 1)

    nblk = B // BLK
    full = lambda shape: pl.BlockSpec(shape, lambda i: (0, 0))
    out, hnew_ext = pl.pallas_call(
        _tc_body,
        grid=(nblk + 1,),
        in_specs=[
            pl.BlockSpec((BLK, D), lambda i: (jnp.minimum(i, nblk - 1), 0)),
            pl.BlockSpec((BLK, LW), lambda i: (jnp.minimum(i, nblk - 1), 0)),
            full((STATE, LW)),
            full((D + UNITS, 3 * UNITS + H1)),
            full((UNITS, UNITS)),
            full((UNITS, H1)),
            full((1, 3 * UNITS + H1)),
            full((1, H1)),
            full((H1, 1)),
            full((1, 1)),
        ],
        out_specs=[
            pl.BlockSpec((BLK, 1), lambda i: (jnp.minimum(i, nblk - 1), 0)),
            pl.BlockSpec((BLK, LW), lambda i: (i, 0)),
        ],
        out_shape=[
            jax.ShapeDtypeStruct((B, 1), jnp.float32),
            jax.ShapeDtypeStruct((HEXT, LW), jnp.float32),
        ],
    )(inputs, h, mem128, wcat, uh, w1h, bcat, b1r, W2, b2r)
    return out, hnew_ext


NSW = 128
NWS = SPAD // NSW
TAIL = STATE - (NWS - 1) * NSW


def _scatter_body(win_hbm, hnew_hbm, newmem_hbm, wbuf_v, idx_v, rows_v, sem):
    wid = _wid()

    @pl.when(wid < NWS)
    def _go():
        s0 = wid * NSW
        pltpu.sync_copy(win_hbm.at[:, pl.ds(s0, NSW)], wbuf_v)
        iota = lax.iota(jnp.int32, 16)
        for g in range(NSW // 16):
            m = wbuf_v[0, pl.ds(g * 16, 16)]
            for t in range(1, NW):
                m = jnp.maximum(m, wbuf_v[t, pl.ds(g * 16, 16)])
            fallback = jnp.full((16,), B + s0 + g * 16, jnp.int32) + iota
            idx_v[pl.ds(g * 16, 16)] = jnp.where(m < 0, fallback, m)
        pltpu.async_copy(hnew_hbm.at[idx_v], rows_v, sem).wait()

        @pl.when(wid < NWS - 1)
        def _full():
            pltpu.sync_copy(rows_v, newmem_hbm.at[pl.ds(s0, NSW)])

        @pl.when(wid == NWS - 1)
        def _tail():
            pltpu.sync_copy(rows_v.at[pl.ds(0, TAIL)],
                            newmem_hbm.at[pl.ds((NWS - 1) * NSW, TAIL)])


@functools.partial(
    pl.kernel,
    out_type=jax.ShapeDtypeStruct((STATE, LW), jnp.float32),
    mesh=_mesh,
    scratch_types=[
        pltpu.VMEM((NW, NSW), jnp.int32),
        pltpu.VMEM((NSW,), jnp.int32),
        pltpu.VMEM((NSW, LW), jnp.float32),
        pltpu.SemaphoreType.DMA,
    ],
    compiler_params=_sc_params,
)
def _sc_scatter(win_hbm, hnew_hbm, newmem_hbm, wbuf_v, idx_v, rows_v, sem):
    _scatter_body(win_hbm, hnew_hbm, newmem_hbm, wbuf_v, idx_v, rows_v, sem)


def kernel(inputs, mem, kernel, rec_kernel, bias, W1, b1, W2, b2):
    ids = jnp.clip(inputs[:, 0].astype(jnp.int32), 0, STATE - 1)
    mem128 = jnp.pad(mem, ((0, 0), (0, LW - UNITS)))
    h, winners = _sc_gather(ids, mem128)
    out, hnew_ext = _dense_stage(inputs, h, mem128, kernel, rec_kernel, bias,
                                 W1, b1, W2, b2)
    new_mem = _sc_scatter(winners, hnew_ext)[:, :UNITS]
    return out, new_mem
